# Initial kernel scaffold; baseline (speedup 1.0000x reference)
import jax
import jax.numpy as jnp
from jax.experimental import pallas as pl


def kernel(selected_traj, road_points, road_mask):
    raise NotImplementedError("write your pallas kernel here")


# speedup vs baseline: 4.0424x; 4.0424x over previous
import functools

import jax
import jax.numpy as jnp
from jax.experimental import pallas as pl


def _proj_kernel(pos_ref, a_ref, b_ref, m_ref, out_ref):
    p = pos_ref[0]
    px = p[:, 0:1]
    py = p[:, 1:2]
    pz = p[:, 2:3]
    ax = a_ref[0, 0:1, :]
    ay = a_ref[0, 1:2, :]
    az = a_ref[0, 2:3, :]
    bx = b_ref[0, 0:1, :]
    by = b_ref[0, 1:2, :]
    bz = b_ref[0, 2:3, :]
    m = m_ref[0]

    dx = bx - ax
    dy = by - ay
    dz = bz - az
    dd = jnp.maximum(dx * dx + dy * dy + dz * dz, 1e-12)

    tn = (px - ax) * dx + (py - ay) * dy + (pz - az) * dz
    t = jnp.clip(tn / dd, 0.0, 1.0)
    qx = ax + t * dx
    qy = ay + t * dy
    qz = az + t * dz
    ex = px - qx
    ey = py - qy
    ez = pz - qz
    dist2 = ex * ex + ey * ey + ez * ez
    dist2 = dist2 + (1.0 - m) * 1e30

    best = jnp.argmin(dist2, axis=1)
    T, NSP = dist2.shape
    onehot = jax.lax.broadcasted_iota(jnp.int32, (T, NSP), 1) == best[:, None]
    qbx = jnp.sum(jnp.where(onehot, qx, 0.0), axis=1, keepdims=True)
    qby = jnp.sum(jnp.where(onehot, qy, 0.0), axis=1, keepdims=True)
    qbz = jnp.sum(jnp.where(onehot, qz, 0.0), axis=1, keepdims=True)

    has_valid = jnp.any(m > 0.0)
    q = jnp.concatenate([qbx, qby, qbz], axis=1)
    out_ref[0] = jnp.where(has_valid, q, p)


@functools.partial(jax.jit, static_argnames=())
def _run(pos, aT, bT, maskf):
    N, T, _ = pos.shape
    NSP = aT.shape[2]
    out = pl.pallas_call(
        _proj_kernel,
        grid=(N,),
        in_specs=[
            pl.BlockSpec((1, T, 3), lambda n: (n, 0, 0)),
            pl.BlockSpec((1, 3, NSP), lambda n: (n, 0, 0)),
            pl.BlockSpec((1, 3, NSP), lambda n: (n, 0, 0)),
            pl.BlockSpec((1, 1, NSP), lambda n: (n, 0, 0)),
        ],
        out_specs=pl.BlockSpec((1, T, 3), lambda n: (n, 0, 0)),
        out_shape=jax.ShapeDtypeStruct((N, T, 3), pos.dtype),
    )(pos, aT, bT, maskf)
    return out


def kernel(selected_traj, road_points, road_mask):
    pos = selected_traj[..., 0:3]
    rest = selected_traj[..., 3:]
    N, NB, NP, D = road_points.shape
    NS = NB * (NP - 1)
    NSP = (NS + 127) // 128 * 128
    pad = NSP - NS

    a = road_points[:, :, :-1, :].reshape(N, NS, D)
    b = road_points[:, :, 1:, :].reshape(N, NS, D)
    seg_mask = (road_mask[:, :, :-1] & road_mask[:, :, 1:]).reshape(N, NS)

    aT = jnp.pad(a.transpose(0, 2, 1), ((0, 0), (0, 0), (0, pad)))
    bT = jnp.pad(b.transpose(0, 2, 1), ((0, 0), (0, 0), (0, pad)))
    maskf = jnp.pad(seg_mask.astype(jnp.float32)[:, None, :],
                    ((0, 0), (0, 0), (0, pad)))

    pos_proj = _run(pos.astype(jnp.float32), aT, bT, maskf)
    if rest.shape[-1] == 0:
        return pos_proj
    return jnp.concatenate([pos_proj, rest], axis=-1)
